# Initial kernel scaffold; baseline (speedup 1.0000x reference)
import jax
import jax.numpy as jnp
from jax.experimental import pallas as pl


def kernel(input_ids, answer_tag_ids, word_table, answer_table):
    raise NotImplementedError("write your pallas kernel here")


# speedup vs baseline: 5.3306x; 5.3306x over previous
import functools

import jax
import jax.numpy as jnp
from jax import lax
from jax.experimental import pallas as pl
from jax.experimental.pallas import tpu as pltpu
from jax.experimental.pallas import tpu_sc as plsc

VOCAB = 100000
EMB = 128
ANS_EMB = 16
OUT_D = EMB + ANS_EMB
B = 1024
L = 200
N = B * L

IDXW = 128
CHUNK = 256
K = CHUNK // IDXW

_GATHER_DNUMS = lax.GatherDimensionNumbers(
    offset_dims=(), collapsed_slice_dims=(0,), start_index_map=(0,))


def _dyn_gather(v, idx):
    return lax.gather(v, idx[:, None], _GATHER_DNUMS, slice_sizes=(1,),
                      mode=lax.GatherScatterMode.PROMISE_IN_BOUNDS)


@functools.lru_cache(maxsize=None)
def _build():
    info = plsc.get_sparse_core_info()
    nc, ns = info.num_cores, info.num_subcores
    nw = nc * ns
    per_w = N // nw
    nchunk = per_w // CHUNK
    assert per_w % CHUNK == 0

    mesh = plsc.VectorSubcoreMesh(core_axis_name="c", subcore_axis_name="s")

    @functools.partial(
        pl.kernel,
        mesh=mesh,
        out_type=jax.ShapeDtypeStruct((N, OUT_D), jnp.float32),
        scratch_types=[
            pltpu.VMEM((CHUNK,), jnp.int32),
            pltpu.VMEM((CHUNK,), jnp.int32),
            pltpu.VMEM((CHUNK, EMB), jnp.float32),
            pltpu.VMEM((CHUNK, ANS_EMB), jnp.float32),
            pltpu.VMEM((64,), jnp.float32),
            pltpu.SemaphoreType.DMA,
        ],
    )
    def emb_kernel(wid_hbm, aid_hbm, word_hbm, ansflat_hbm, out_hbm,
                   widx, aidx, wrows, arows, anstab, wsem):
        w = lax.axis_index("s") * nc + lax.axis_index("c")
        row0 = w * per_w
        pltpu.sync_copy(ansflat_hbm, anstab)
        arow_vecs = [anstab[pl.ds(r * ANS_EMB, 16)] for r in range(4)]

        def chunk_body(ci, _):
            base = row0 + ci * CHUNK
            pltpu.sync_copy(wid_hbm.at[pl.ds(base, CHUNK)], widx)
            pltpu.sync_copy(aid_hbm.at[pl.ds(base, CHUNK)], aidx)
            for j in range(K):
                sl = pl.ds(j * IDXW, IDXW)
                pltpu.async_copy(word_hbm.at[widx.at[sl]], wrows.at[sl], wsem)

            def ans_body(g, _):
                r0 = g * 16
                tf = aidx[pl.ds(r0, 16)].astype(jnp.float32)
                t1, t2, t3 = tf - 1.0, tf - 2.0, tf - 3.0
                d = [t1 * t2 * t3 * (-1.0 / 6.0),
                     tf * t2 * t3 * 0.5,
                     tf * t1 * t3 * (-0.5),
                     tf * t1 * t2 * (1.0 / 6.0)]
                for j in range(16):
                    lane = jnp.full((16,), j, jnp.int32)
                    vals = (_dyn_gather(d[0], lane) * arow_vecs[0]
                            + _dyn_gather(d[1], lane) * arow_vecs[1]
                            + _dyn_gather(d[2], lane) * arow_vecs[2]
                            + _dyn_gather(d[3], lane) * arow_vecs[3])
                    arows[r0 + j, :] = vals
                return ()

            lax.fori_loop(0, CHUNK // 16, ans_body, ())

            for j in range(K):
                sl = pl.ds(j * IDXW, IDXW)
                pltpu.make_async_copy(word_hbm.at[widx.at[sl]],
                                      wrows.at[sl], wsem).wait()
            pltpu.sync_copy(wrows, out_hbm.at[pl.ds(base, CHUNK), pl.ds(0, EMB)])
            pltpu.sync_copy(arows, out_hbm.at[pl.ds(base, CHUNK), pl.ds(EMB, ANS_EMB)])
            return ()

        lax.fori_loop(0, nchunk, chunk_body, ())

    return emb_kernel


def kernel(input_ids, answer_tag_ids, word_table, answer_table):
    wid = input_ids.reshape(N).astype(jnp.int32)
    aid = answer_tag_ids.reshape(N).astype(jnp.int32)
    ansflat = answer_table.reshape(64)
    out = _build()(wid, aid, word_table, ansflat)
    return out.reshape(B, L, OUT_D)
